# baseline (speedup 1.0000x reference)
import functools

import jax
import jax.numpy as jnp
from jax import lax
from jax.experimental import pallas as pl
from jax.experimental.pallas import tpu as pltpu
from jax.experimental.pallas import tpu_sc as plsc

H = 1024
CAP = 16384
B = 2048
K = 16

RBLK = 2048
BQ = 256
CBLK = 2048
NCHUNK = CAP // CBLK
BQ2 = 128
NW = 32
NIDX = B * K
NBI = NIDX // NW
CH = 64


def _keff_body(store_ref, wk_ref, imp_ref, ts_ref, kn_ref, w_ref, wsum_ref,
               sb_ref):
    i = pl.program_id(0)
    keys = lax.dot_general(
        store_ref[...], wk_ref[...], (((1,), (1,)), ((), ())),
        preferred_element_type=jnp.float32)
    n = jnp.sqrt(jnp.sum(keys * keys, axis=1, keepdims=True))
    kn_ref[...] = keys / jnp.maximum(n, 1e-12)
    sb_ref[...] = store_ref[...].astype(jnp.bfloat16)
    ages = 1.0 - ts_ref[...]
    rec = jnp.exp(-jnp.abs(ages) * 0.01)
    w = rec * (imp_ref[...] + 1.0)
    w_ref[...] = w

    @pl.when(i == 0)
    def _():
        wsum_ref[0, 0] = 0.0

    wsum_ref[0, 0] += jnp.sum(w)


def _keff_call(store, W_k, imp2, ts2, interpret=False):
    grid = (CAP // RBLK,)
    return pl.pallas_call(
        _keff_body,
        grid=grid,
        in_specs=[
            pl.BlockSpec((RBLK, H), lambda i: (i, 0)),
            pl.BlockSpec((H, H), lambda i: (0, 0)),
            pl.BlockSpec((1, RBLK), lambda i: (0, i)),
            pl.BlockSpec((1, RBLK), lambda i: (0, i)),
        ],
        out_specs=(
            pl.BlockSpec((RBLK, H), lambda i: (i, 0)),
            pl.BlockSpec((1, RBLK), lambda i: (0, i)),
            pl.BlockSpec(memory_space=pltpu.SMEM),
            pl.BlockSpec((RBLK, H), lambda i: (i, 0)),
        ),
        out_shape=(
            jax.ShapeDtypeStruct((CAP, H), jnp.float32),
            jax.ShapeDtypeStruct((1, CAP), jnp.float32),
            jax.ShapeDtypeStruct((1, 1), jnp.float32),
            jax.ShapeDtypeStruct((CAP, H), jnp.bfloat16),
        ),
        interpret=interpret,
    )(store, W_k, imp2, ts2)


def _topk_body(s_ref, q_ref, kn_ref, w_ref, idx_ref, attn_ref,
               cv_scr, ci_scr):
    j = pl.program_id(1)
    q = q_ref[...]
    qn = q / jnp.maximum(
        jnp.sqrt(jnp.sum(q * q, axis=1, keepdims=True)), 1e-12)
    sim = lax.dot_general(
        qn, kn_ref[...], (((1,), (1,)), ((), ())),
        preferred_element_type=jnp.float32)
    sim = sim * (w_ref[...] * s_ref[0, 0])

    cols = lax.broadcasted_iota(jnp.int32, (BQ, CBLK), 1)
    lv = []
    li = []
    for _t in range(K):
        m = jnp.max(sim, axis=1, keepdims=True)
        im = jnp.min(jnp.where(sim == m, cols, CBLK), axis=1, keepdims=True)
        lv.append(m)
        li.append(im)
        sim = jnp.where(cols == im, -jnp.inf, sim)
    pad_v = jnp.full((BQ, 128 - K), -jnp.inf, jnp.float32)
    pad_i = jnp.full((BQ, 128 - K), CAP, jnp.int32)
    off = pl.multiple_of(j * 128, 128)
    cv_scr[:, pl.ds(off, 128)] = jnp.concatenate(lv + [pad_v], axis=1)
    ci_scr[:, pl.ds(off, 128)] = jnp.concatenate(
        [x + j * CBLK for x in li] + [pad_i], axis=1)

    @pl.when(j == NCHUNK - 1)
    def _():
        cand_v = cv_scr[...]
        cand_i = ci_scr[...]
        vals = []
        idxs = []
        for _t in range(K):
            m = jnp.max(cand_v, axis=1, keepdims=True)
            hit = cand_v == m
            gi = jnp.min(jnp.where(hit, cand_i, CAP), axis=1, keepdims=True)
            vals.append(m)
            idxs.append(gi)
            cand_v = jnp.where(hit & (cand_i == gi), -jnp.inf, cand_v)
        v = jnp.concatenate(vals, axis=1)
        mx = jnp.max(v, axis=1, keepdims=True)
        e = jnp.exp(v - mx)
        attn_ref[...] = e / jnp.sum(e, axis=1, keepdims=True)
        idx_ref[...] = jnp.concatenate(idxs, axis=1)


def _topk_call(s, query, kn, w, interpret=False):
    grid = (B // BQ, NCHUNK)
    return pl.pallas_call(
        _topk_body,
        grid=grid,
        in_specs=[
            pl.BlockSpec(memory_space=pltpu.SMEM),
            pl.BlockSpec((BQ, H), lambda i, j: (i, 0)),
            pl.BlockSpec((CBLK, H), lambda i, j: (j, 0)),
            pl.BlockSpec((1, CBLK), lambda i, j: (0, j)),
        ],
        out_specs=(
            pl.BlockSpec((BQ, K), lambda i, j: (i, 0)),
            pl.BlockSpec((BQ, K), lambda i, j: (i, 0)),
        ),
        out_shape=(
            jax.ShapeDtypeStruct((B, K), jnp.int32),
            jax.ShapeDtypeStruct((B, K), jnp.float32),
        ),
        scratch_shapes=[pltpu.VMEM((BQ, NCHUNK * 128), jnp.float32),
                        pltpu.VMEM((BQ, NCHUNK * 128), jnp.int32)],
        interpret=interpret,
    )(s, query, kn, w)


def _sc_gather_body(store_hbm, idx_hbm, out_hbm, idx_v, rows_v, sem):
    wid = lax.axis_index("s") * 2 + lax.axis_index("c")
    base = wid * NBI
    pltpu.sync_copy(idx_hbm.at[pl.ds(base, NBI)], idx_v)
    for c in range(NBI // CH):
        pltpu.async_copy(
            store_hbm.at[idx_v.at[pl.ds(c * CH, CH)]], rows_v, sem).wait()
        pltpu.sync_copy(rows_v, out_hbm.at[pl.ds(base + c * CH, CH)])


def _gather_call(store, idx_flat):
    mesh = plsc.VectorSubcoreMesh(core_axis_name="c", subcore_axis_name="s")
    fn = functools.partial(
        pl.kernel,
        mesh=mesh,
        out_type=jax.ShapeDtypeStruct((NIDX, H // 2), jnp.int32),
        scratch_types=[
            pltpu.VMEM((NBI,), jnp.int32),
            pltpu.VMEM((CH, H // 2), jnp.int32),
            pltpu.SemaphoreType.DMA,
        ],
    )(_sc_gather_body)
    return fn(store, idx_flat)


def _combine_body(g_ref, a_ref, wv_ref, wo_ref, o_ref):
    a = a_ref[...]
    acc = a[:, 0:1] * g_ref[0].astype(jnp.float32)
    for k in range(1, K):
        acc = acc + a[:, k:k + 1] * g_ref[k].astype(jnp.float32)
    t = lax.dot_general(
        acc, wv_ref[...], (((1,), (1,)), ((), ())),
        preferred_element_type=jnp.float32)
    o_ref[...] = lax.dot_general(
        t, wo_ref[...], (((1,), (1,)), ((), ())),
        preferred_element_type=jnp.float32)


def _combine_call(g, attn, W_v, W_o, interpret=False):
    grid = (B // BQ2,)
    return pl.pallas_call(
        _combine_body,
        grid=grid,
        in_specs=[
            pl.BlockSpec((K, BQ2, H), lambda i: (0, i, 0)),
            pl.BlockSpec((BQ2, K), lambda i: (i, 0)),
            pl.BlockSpec((H, H), lambda i: (0, 0)),
            pl.BlockSpec((H, H), lambda i: (0, 0)),
        ],
        out_specs=pl.BlockSpec((BQ2, H), lambda i: (i, 0)),
        out_shape=jax.ShapeDtypeStruct((B, H), jnp.float32),
        interpret=interpret,
    )(g, attn, W_v, W_o)


def kernel(query, store, importance, timestamps, W_k, W_v, W_o):
    imp2 = importance.reshape(1, CAP)
    ts2 = timestamps.reshape(1, CAP)
    kn, w, wsum, store_bf = _keff_call(store, W_k, imp2, ts2)
    s = (1.0 / (wsum + 1e-8)).reshape(1, 1)
    idx, attn = _topk_call(s, query, kn, w)
    idx_km = idx.T.reshape(-1)
    store_i32 = lax.bitcast_convert_type(
        store_bf.reshape(CAP, H // 2, 2), jnp.int32)
    g_flat = _gather_call(store_i32, idx_km)
    g = lax.bitcast_convert_type(g_flat, jnp.bfloat16).reshape(K, B, H)
    return _combine_call(g, attn, W_v, W_o)


# speedup vs baseline: 1.7754x; 1.7754x over previous
import functools

import jax
import jax.numpy as jnp
from jax import lax
from jax.experimental import pallas as pl
from jax.experimental.pallas import tpu as pltpu
from jax.experimental.pallas import tpu_sc as plsc

H = 1024
CAP = 16384
B = 2048
K = 16

RBLK = 2048
BQ = 256
CBLK = 2048
NCHUNK = CAP // CBLK
BQ2 = 128
NW = 32
NIDX = B * K
NBI = NIDX // NW
CH = 64


def _keff_body(store_ref, wk_ref, imp_ref, ts_ref, kn_ref, w_ref, wsum_ref):
    i = pl.program_id(0)
    keys = lax.dot_general(
        store_ref[...], wk_ref[...], (((1,), (1,)), ((), ())),
        preferred_element_type=jnp.float32)
    n = jnp.sqrt(jnp.sum(keys * keys, axis=1, keepdims=True))
    kn_ref[...] = keys / jnp.maximum(n, 1e-12)
    ages = 1.0 - ts_ref[...]
    rec = jnp.exp(-jnp.abs(ages) * 0.01)
    w = rec * (imp_ref[...] + 1.0)
    w_ref[...] = w

    @pl.when(i == 0)
    def _():
        wsum_ref[0, 0] = 0.0

    wsum_ref[0, 0] += jnp.sum(w)


def _keff_call(store, W_k, imp2, ts2, interpret=False):
    grid = (CAP // RBLK,)
    return pl.pallas_call(
        _keff_body,
        grid=grid,
        in_specs=[
            pl.BlockSpec((RBLK, H), lambda i: (i, 0)),
            pl.BlockSpec((H, H), lambda i: (0, 0)),
            pl.BlockSpec((1, RBLK), lambda i: (0, i)),
            pl.BlockSpec((1, RBLK), lambda i: (0, i)),
        ],
        out_specs=(
            pl.BlockSpec((RBLK, H), lambda i: (i, 0)),
            pl.BlockSpec((1, RBLK), lambda i: (0, i)),
            pl.BlockSpec(memory_space=pltpu.SMEM),
        ),
        out_shape=(
            jax.ShapeDtypeStruct((CAP, H), jnp.float32),
            jax.ShapeDtypeStruct((1, CAP), jnp.float32),
            jax.ShapeDtypeStruct((1, 1), jnp.float32),
        ),
        interpret=interpret,
    )(store, W_k, imp2, ts2)


def _topk_body(s_ref, q_ref, kn_ref, w_ref, idx_ref, attn_ref,
               cv_scr, ci_scr):
    j = pl.program_id(1)
    q = q_ref[...]
    qn = q / jnp.maximum(
        jnp.sqrt(jnp.sum(q * q, axis=1, keepdims=True)), 1e-12)
    sim = lax.dot_general(
        qn, kn_ref[...], (((1,), (1,)), ((), ())),
        preferred_element_type=jnp.float32)
    sim = sim * (w_ref[...] * s_ref[0, 0])

    cols = lax.broadcasted_iota(jnp.int32, (BQ, CBLK), 1)
    lv = []
    li = []
    for _t in range(K):
        m = jnp.max(sim, axis=1, keepdims=True)
        im = jnp.min(jnp.where(sim == m, cols, CBLK), axis=1, keepdims=True)
        lv.append(m)
        li.append(im)
        sim = jnp.where(cols == im, -jnp.inf, sim)
    pad_v = jnp.full((BQ, 128 - K), -jnp.inf, jnp.float32)
    pad_i = jnp.full((BQ, 128 - K), CAP, jnp.int32)
    off = pl.multiple_of(j * 128, 128)
    cv_scr[:, pl.ds(off, 128)] = jnp.concatenate(lv + [pad_v], axis=1)
    ci_scr[:, pl.ds(off, 128)] = jnp.concatenate(
        [x + j * CBLK for x in li] + [pad_i], axis=1)

    @pl.when(j == NCHUNK - 1)
    def _():
        cand_v = cv_scr[...]
        cand_i = ci_scr[...]
        vals = []
        idxs = []
        for _t in range(K):
            m = jnp.max(cand_v, axis=1, keepdims=True)
            hit = cand_v == m
            gi = jnp.min(jnp.where(hit, cand_i, CAP), axis=1, keepdims=True)
            vals.append(m)
            idxs.append(gi)
            cand_v = jnp.where(hit & (cand_i == gi), -jnp.inf, cand_v)
        v = jnp.concatenate(vals, axis=1)
        mx = jnp.max(v, axis=1, keepdims=True)
        e = jnp.exp(v - mx)
        attn_ref[...] = e / jnp.sum(e, axis=1, keepdims=True)
        idx_ref[...] = jnp.concatenate(idxs, axis=1)


def _topk_call(s, query, kn, w, interpret=False):
    grid = (B // BQ, NCHUNK)
    return pl.pallas_call(
        _topk_body,
        grid=grid,
        in_specs=[
            pl.BlockSpec(memory_space=pltpu.SMEM),
            pl.BlockSpec((BQ, H), lambda i, j: (i, 0)),
            pl.BlockSpec((CBLK, H), lambda i, j: (j, 0)),
            pl.BlockSpec((1, CBLK), lambda i, j: (0, j)),
        ],
        out_specs=(
            pl.BlockSpec((BQ, K), lambda i, j: (i, 0)),
            pl.BlockSpec((BQ, K), lambda i, j: (i, 0)),
        ),
        out_shape=(
            jax.ShapeDtypeStruct((B, K), jnp.int32),
            jax.ShapeDtypeStruct((B, K), jnp.float32),
        ),
        scratch_shapes=[pltpu.VMEM((BQ, NCHUNK * 128), jnp.float32),
                        pltpu.VMEM((BQ, NCHUNK * 128), jnp.int32)],
        interpret=interpret,
    )(s, query, kn, w)


def _sc_gather_body(store_hbm, idx_hbm, out_hbm, idx_v, rows_v, sem):
    wid = lax.axis_index("s") * 2 + lax.axis_index("c")
    base = wid * NBI
    pltpu.sync_copy(idx_hbm.at[pl.ds(base, NBI)], idx_v)
    for c in range(NBI // CH):
        pltpu.async_copy(
            store_hbm.at[idx_v.at[pl.ds(c * CH, CH)]], rows_v, sem).wait()
        pltpu.sync_copy(rows_v, out_hbm.at[pl.ds(base + c * CH, CH)])


def _gather_call(store, idx_flat):
    mesh = plsc.VectorSubcoreMesh(core_axis_name="c", subcore_axis_name="s")
    fn = functools.partial(
        pl.kernel,
        mesh=mesh,
        out_type=jax.ShapeDtypeStruct((NIDX, H), jnp.float32),
        scratch_types=[
            pltpu.VMEM((NBI,), jnp.int32),
            pltpu.VMEM((CH, H), jnp.float32),
            pltpu.SemaphoreType.DMA,
        ],
    )(_sc_gather_body)
    return fn(store, idx_flat)


def _combine_body(g_ref, a_ref, wv_ref, wo_ref, o_ref):
    a = a_ref[...]
    acc = a[:, 0:1] * g_ref[0]
    for k in range(1, K):
        acc = acc + a[:, k:k + 1] * g_ref[k]
    t = lax.dot_general(
        acc, wv_ref[...], (((1,), (1,)), ((), ())),
        preferred_element_type=jnp.float32)
    o_ref[...] = lax.dot_general(
        t, wo_ref[...], (((1,), (1,)), ((), ())),
        preferred_element_type=jnp.float32)


def _combine_call(g, attn, W_v, W_o, interpret=False):
    grid = (B // BQ2,)
    return pl.pallas_call(
        _combine_body,
        grid=grid,
        in_specs=[
            pl.BlockSpec((K, BQ2, H), lambda i: (0, i, 0)),
            pl.BlockSpec((BQ2, K), lambda i: (i, 0)),
            pl.BlockSpec((H, H), lambda i: (0, 0)),
            pl.BlockSpec((H, H), lambda i: (0, 0)),
        ],
        out_specs=pl.BlockSpec((BQ2, H), lambda i: (i, 0)),
        out_shape=jax.ShapeDtypeStruct((B, H), jnp.float32),
        interpret=interpret,
    )(g, attn, W_v, W_o)


def kernel(query, store, importance, timestamps, W_k, W_v, W_o):
    imp2 = importance.reshape(1, CAP)
    ts2 = timestamps.reshape(1, CAP)
    kn, w, wsum = _keff_call(store, W_k, imp2, ts2)
    s = (1.0 / (wsum + 1e-8)).reshape(1, 1)
    idx, attn = _topk_call(s, query, kn, w)
    idx_km = idx.T.reshape(-1)
    g_flat = _gather_call(store, idx_km)
    g = g_flat.reshape(K, B, H)
    return _combine_call(g, attn, W_v, W_o)
